# baseline (speedup 1.0000x reference)
import functools

import jax
import jax.numpy as jnp
import numpy as np
from jax import lax
from jax.experimental import pallas as pl
from jax.experimental.pallas import tpu as pltpu
from jax.experimental.pallas import tpu_sc as plsc

_TEMP = np.float32(0.7)
_NC, _NS = 2, 16
_NW = _NC * _NS
_L = 16


def _sc_gate_call(h, w, T, D, E, CH, TBLK):
    WT = T // _NW
    n_chunks = WT // CH
    n_pairs = n_chunks // 2
    KC = D // _L
    assert CH % TBLK == 0 and (TBLK * E) % _L == 0

    mesh = plsc.VectorSubcoreMesh(core_axis_name="c", subcore_axis_name="s")

    _dnums = lax.GatherDimensionNumbers(
        offset_dims=(), collapsed_slice_dims=(0,), start_index_map=(0,)
    )

    def _perm(v, idx2d):
        return lax.gather(
            v, idx2d, _dnums, (1,),
            mode=lax.GatherScatterMode.PROMISE_IN_BOUNDS,
        )

    def compute_chunk(buf, w_v, out_v, c, lane, xor_idx):
        for t0 in range(0, CH, TBLK):
            def kbody(k, accs):
                hs = [buf[t0 + i, pl.ds(k * _L, _L)] for i in range(TBLK)]
                new = []
                for e in range(E):
                    wv = w_v[e, pl.ds(k * _L, _L)]
                    for i in range(TBLK):
                        new.append(accs[e * TBLK + i] + hs[i] * wv)
                return tuple(new)

            zero = jnp.zeros((_L,), jnp.float32)
            accs = lax.fori_loop(
                0, KC, kbody, tuple([zero] * (E * TBLK)), unroll=2
            )
            tots = [None] * (TBLK * E)
            for e in range(E):
                for i in range(TBLK):
                    v = accs[e * TBLK + i]
                    for sh in range(4):
                        v = v + _perm(v, xor_idx[sh])
                    tots[i * E + e] = v
            for g in range(TBLK * E // _L):
                res = tots[g * _L]
                for l in range(1, _L):
                    res = jnp.where(lane == l, tots[g * _L + l], res)
                off = (c * CH + t0) * E + g * _L
                out_v[pl.ds(off, _L)] = res

    @functools.partial(
        pl.kernel,
        out_type=jax.ShapeDtypeStruct((T * E,), jnp.float32),
        mesh=mesh,
        scratch_types=[
            pltpu.VMEM((E, D), jnp.float32),
            pltpu.VMEM((2, CH, D), jnp.float32),
            pltpu.VMEM((WT * E,), jnp.float32),
            pltpu.SemaphoreType.DMA,
            pltpu.SemaphoreType.DMA,
        ],
    )
    def k(h_hbm, w_hbm, out_hbm, w_v, buf_v, out_v, sem0, sem1):
        wid = lax.axis_index("s") * _NC + lax.axis_index("c")
        base = wid * WT
        lane = lax.iota(jnp.int32, _L)
        xor_idx = [(lane ^ (1 << sh)).reshape(_L, 1) for sh in range(4)]
        pltpu.sync_copy(w_hbm, w_v)
        sems = (sem0, sem1)

        def start(c, b):
            pltpu.async_copy(
                h_hbm.at[pl.ds(base + c * CH, CH)], buf_v.at[b], sems[b]
            )

        def wait(b):
            pltpu.make_async_copy(
                h_hbm.at[pl.ds(base, CH)], buf_v.at[b], sems[b]
            ).wait()

        start(0, 0)
        if n_chunks > 1:
            start(1, 1)

        def pair_body(p, carry):
            c0 = 2 * p
            wait(0)
            compute_chunk(buf_v.at[0], w_v, out_v, c0, lane, xor_idx)

            @pl.when(c0 + 2 < n_chunks)
            def _():
                start(c0 + 2, 0)

            @pl.when(c0 + 1 < n_chunks)
            def _():
                wait(1)
                compute_chunk(buf_v.at[1], w_v, out_v, c0 + 1, lane, xor_idx)

                @pl.when(c0 + 3 < n_chunks)
                def _():
                    start(c0 + 3, 1)

            return carry

        lax.fori_loop(0, n_pairs, pair_body, 0)
        pltpu.sync_copy(out_v, out_hbm.at[pl.ds(base * E, WT * E)])

    return k(h, w)


def kernel(hidden_states, gate_weight):
    B, S, D = hidden_states.shape
    E = gate_weight.shape[0]
    T = B * S
    h = hidden_states.reshape(T, D)
    w_scaled = gate_weight / _TEMP
    out = _sc_gate_call(h, w_scaled, T, D, E, CH=16, TBLK=4)
    return out.reshape(B, S, E)


# speedup vs baseline: 3.9189x; 3.9189x over previous
import functools

import jax
import jax.numpy as jnp
import numpy as np
from jax import lax
from jax.experimental import pallas as pl
from jax.experimental.pallas import tpu as pltpu
from jax.experimental.pallas import tpu_sc as plsc

_TEMP = np.float32(0.7)
_NC, _NS = 2, 16
_NW = _NC * _NS
_L = 16


def _sc_gate_call(h, w, t_off, T_SC, D, E, CH, TBLK):
    WT = T_SC // _NW
    n_chunks = WT // CH
    n_pairs = n_chunks // 2
    KC = D // _L
    assert WT % CH == 0 and n_chunks % 2 == 0 and CH % TBLK == 0
    assert (TBLK * E) % _L == 0

    mesh = plsc.VectorSubcoreMesh(core_axis_name="c", subcore_axis_name="s")

    _dnums = lax.GatherDimensionNumbers(
        offset_dims=(), collapsed_slice_dims=(0,), start_index_map=(0,)
    )

    def _perm(v, idx2d):
        return lax.gather(
            v, idx2d, _dnums, (1,),
            mode=lax.GatherScatterMode.PROMISE_IN_BOUNDS,
        )

    def compute_chunk(buf, w_v, out_v, c, lane, xor_idx):
        for t0 in range(0, CH, TBLK):
            def kbody(k, accs):
                hs = [buf[t0 + i, pl.ds(k * _L, _L)] for i in range(TBLK)]
                new = []
                for e in range(E):
                    wv = w_v[e, pl.ds(k * _L, _L)]
                    for i in range(TBLK):
                        new.append(accs[e * TBLK + i] + hs[i] * wv)
                return tuple(new)

            zero = jnp.zeros((_L,), jnp.float32)
            accs = lax.fori_loop(
                0, KC, kbody, tuple([zero] * (E * TBLK)), unroll=2
            )
            tots = [None] * (TBLK * E)
            for e in range(E):
                for i in range(TBLK):
                    v = accs[e * TBLK + i]
                    for sh in range(4):
                        v = v + _perm(v, xor_idx[sh])
                    tots[i * E + e] = v
            for g in range(TBLK * E // _L):
                res = tots[g * _L]
                for l in range(1, _L):
                    res = jnp.where(lane == l, tots[g * _L + l], res)
                off = (c * CH + t0) * E + g * _L
                out_v[pl.ds(off, _L)] = res

    @functools.partial(
        pl.kernel,
        out_type=jax.ShapeDtypeStruct((T_SC * E,), jnp.float32),
        mesh=mesh,
        scratch_types=[
            pltpu.VMEM((E, D), jnp.float32),
            pltpu.VMEM((2, CH, D), jnp.float32),
            pltpu.VMEM((WT * E,), jnp.float32),
            pltpu.SemaphoreType.DMA,
            pltpu.SemaphoreType.DMA,
        ],
    )
    def k(h_hbm, w_hbm, out_hbm, w_v, buf_v, out_v, sem0, sem1):
        wid = lax.axis_index("s") * _NC + lax.axis_index("c")
        base = t_off + wid * WT
        lane = lax.iota(jnp.int32, _L)
        xor_idx = [(lane ^ (1 << sh)).reshape(_L, 1) for sh in range(4)]
        pltpu.sync_copy(w_hbm, w_v)
        sems = (sem0, sem1)

        def start(c, b):
            pltpu.async_copy(
                h_hbm.at[pl.ds(base + c * CH, CH)], buf_v.at[b], sems[b]
            )

        def wait(b):
            pltpu.make_async_copy(
                h_hbm.at[pl.ds(base, CH)], buf_v.at[b], sems[b]
            ).wait()

        start(0, 0)
        if n_chunks > 1:
            start(1, 1)

        def pair_body(p, carry):
            c0 = 2 * p
            wait(0)
            compute_chunk(buf_v.at[0], w_v, out_v, c0, lane, xor_idx)

            @pl.when(c0 + 2 < n_chunks)
            def _():
                start(c0 + 2, 0)

            wait(1)
            compute_chunk(buf_v.at[1], w_v, out_v, c0 + 1, lane, xor_idx)

            @pl.when(c0 + 3 < n_chunks)
            def _():
                start(c0 + 3, 1)

            return carry

        lax.fori_loop(0, n_pairs, pair_body, 0)
        pltpu.sync_copy(out_v, out_hbm.at[pl.ds(wid * WT * E, WT * E)])

    return k(h, w)


def _tc_body(h_ref, w_ref, o_ref):
    o_ref[...] = jnp.dot(
        h_ref[...], w_ref[...], preferred_element_type=jnp.float32
    )


def _tc_gate_call(h, wT, t_off, T_TC, D, E, TB):
    nblk = t_off // TB
    return pl.pallas_call(
        _tc_body,
        grid=(T_TC // TB,),
        in_specs=[
            pl.BlockSpec((TB, D), lambda i: (i + nblk, 0)),
            pl.BlockSpec((D, E), lambda i: (0, 0)),
        ],
        out_specs=pl.BlockSpec((TB, E), lambda i: (i, 0)),
        out_shape=jax.ShapeDtypeStruct((T_TC, E), jnp.float32),
    )(h, wT)


def kernel(hidden_states, gate_weight):
    B, S, D = hidden_states.shape
    E = gate_weight.shape[0]
    T = B * S
    h = hidden_states.reshape(T, D)
    w_scaled = gate_weight / _TEMP

    T_SC = 1024
    out_sc = _sc_gate_call(h, w_scaled, T - T_SC, T_SC, D, E, CH=16, TBLK=4)
    out_tc = _tc_gate_call(h, w_scaled.T, 0, T - T_SC, D, E, TB=1024)
    out = jnp.concatenate([out_tc, out_sc.reshape(T_SC, E)], axis=0)
    return out.reshape(B, S, E)


# speedup vs baseline: 9.2489x; 2.3601x over previous
import functools

import jax
import jax.numpy as jnp
import numpy as np
from jax import lax
from jax.experimental import pallas as pl
from jax.experimental.pallas import tpu as pltpu
from jax.experimental.pallas import tpu_sc as plsc

_TEMP = np.float32(0.7)
_INV_TEMP = np.float32(1.0) / _TEMP


def _tc_manual(h, w, t_off, T_TC, D, E, RB, NBUF):
    n_blk = T_TC // RB
    assert n_blk % NBUF == 0

    def body(h_hbm, w_ref, o_ref, bufs, sems):
        def start(b, s):
            pltpu.make_async_copy(
                h_hbm.at[pl.ds(t_off + b * RB, RB)], bufs.at[s], sems.at[s]
            ).start()

        def wait(s):
            pltpu.make_async_copy(
                h_hbm.at[pl.ds(t_off, RB)], bufs.at[s], sems.at[s]
            ).wait()

        for s in range(NBUF):
            start(s, s)
        w = w_ref[...]

        def grp_body(g, carry):
            b0 = g * NBUF
            for s in range(NBUF):
                wait(s)
                r = lax.dot_general(
                    w, bufs[s],
                    (((1,), (1,)), ((), ())),
                    preferred_element_type=jnp.float32,
                )
                o_ref[:, pl.ds((b0 + s) * RB, RB)] = r * _INV_TEMP

                @pl.when(b0 + s + NBUF < n_blk)
                def _():
                    start(b0 + s + NBUF, s)

            return carry

        lax.fori_loop(0, n_blk // NBUF, grp_body, 0)

    return pl.pallas_call(
        body,
        in_specs=[
            pl.BlockSpec(memory_space=pl.ANY),
            pl.BlockSpec((E, D), lambda: (0, 0)),
        ],
        out_specs=pl.BlockSpec((E, T_TC), lambda: (0, 0)),
        out_shape=jax.ShapeDtypeStruct((E, T_TC), jnp.float32),
        scratch_shapes=[
            pltpu.VMEM((NBUF, RB, D), jnp.float32),
            pltpu.SemaphoreType.DMA((NBUF,)),
        ],
    )(h, w)


def kernel(hidden_states, gate_weight):
    B, S, D = hidden_states.shape
    E = gate_weight.shape[0]
    T = B * S
    h = hidden_states.reshape(T, D)
    out = _tc_manual(h, gate_weight, 0, T, D, E, RB=256, NBUF=4)
    return out.T.reshape(B, S, E)
